# baseline (speedup 1.0000x reference)
import jax
import jax.numpy as jnp
from jax.experimental import pallas as pl
from jax.experimental.pallas import tpu as pltpu

SPIKE_THRESHOLD = 0.01


_BT_BLK = 8


def _fused_panel(x_ref, w_ref, b_ref, o_ref):
    for p in range(x_ref.shape[0]):
        xb = x_ref[p]
        y = jnp.dot(w_ref[...], xb, preferred_element_type=jnp.float32) + b_ref[...]
        peak = jnp.max(jnp.abs(xb), axis=0, keepdims=True)
        o_ref[p] = jnp.where(peak > SPIKE_THRESHOLD, y, 0.0)


def kernel(x, W, b):
    B, T, S, D = x.shape
    nbt = B * T
    xt = x.transpose(0, 1, 3, 2).reshape(nbt, D, S)
    out_t = pl.pallas_call(
        _fused_panel,
        grid=(nbt // _BT_BLK,),
        in_specs=[
            pl.BlockSpec((_BT_BLK, D, S), lambda i: (i, 0, 0)),
            pl.BlockSpec((D, D), lambda i: (0, 0)),
            pl.BlockSpec((D, 1), lambda i: (0, 0)),
        ],
        out_specs=pl.BlockSpec((_BT_BLK, D, S), lambda i: (i, 0, 0)),
        out_shape=jax.ShapeDtypeStruct((nbt, D, S), x.dtype),
        compiler_params=pltpu.CompilerParams(vmem_limit_bytes=128 * 1024 * 1024),
    )(xt, W, b.reshape(D, 1))
    return out_t.reshape(B, T, D, S).transpose(0, 1, 3, 2)
